# baseline (speedup 1.0000x reference)
import jax
import jax.numpy as jnp
from jax import lax
from jax.experimental import pallas as pl
from jax.experimental.layout import Format, Layout
from jax.experimental.pallas import tpu as pltpu
from jax.experimental.pallas import tpu_sc as plsc

B = 16384
NUM_COLS = 39
N_NUM = 13
N_CAT = 26
D = 32
OUT_COLS = N_NUM + N_CAT * D
CAT_STRIDE = 100000

NC, NS = 2, 16
NW = NC * NS
RW = B // NW
CHUNK = 8
NCHUNK = RW // CHUNK
IDX_PER_CHUNK = CHUNK * N_CAT
NUM_VECS = -(-CHUNK * N_NUM // 16)
EMB_VECS = N_CAT * D // 16


def _body(x_ref, table_ref, out_ref, xbuf, idxbuf, gstage, rowbuf, sem):
    wid = lax.axis_index("s") * NC + lax.axis_index("c")
    lanes = lax.iota(jnp.int32, 16)
    zeros = jnp.zeros((16,), jnp.int32)

    def chunk_body(k, carry):
        base = wid * RW + k * CHUNK
        pltpu.sync_copy(x_ref.at[pl.ds(base, CHUNK), :], xbuf)

        r = zeros
        j = lanes
        for s in range(IDX_PER_CHUNK // 16):
            v = plsc.load_gather(xbuf, [r, j + N_NUM])
            idxbuf[pl.ds(s * 16, 16)] = v.astype(jnp.int32) + j * CAT_STRIDE
            t1 = j + 16
            w = t1 >= N_CAT
            r = jnp.where(w, r + 1, r)
            j = jnp.where(w, t1 - N_CAT, t1)

        cp = pltpu.async_copy(table_ref.at[idxbuf.at[:]], gstage.at[:, :], sem)

        w0 = lanes >= N_NUM
        r = jnp.where(w0, zeros + 1, zeros)
        c = jnp.where(w0, lanes - N_NUM, lanes)
        for _ in range(NUM_VECS):
            v = plsc.load_gather(xbuf, [r, c])
            plsc.store_scatter(rowbuf, [r, c], v)
            t1 = c + (16 - N_NUM)
            w = t1 >= N_NUM
            r = jnp.where(w, r + 2, r + 1)
            c = jnp.where(w, t1 - N_NUM, t1)
            r = jnp.where(r >= CHUNK, r - CHUNK, r)

        cp.wait()

        def pack_row(r, carry):
            rv = zeros + r
            for t in range(EMB_VECS):
                v = plsc.load_gather(
                    gstage, [zeros + (N_CAT * r + t // 2), (t % 2) * 16 + lanes])
                plsc.store_scatter(rowbuf, [rv, N_NUM + t * 16 + lanes], v)
            return carry

        lax.fori_loop(0, CHUNK, pack_row, 0)

        pltpu.sync_copy(rowbuf, out_ref.at[pl.ds(base, CHUNK), :])
        return carry

    lax.fori_loop(0, NCHUNK, chunk_body, 0)


def _run(x, table):
    run = pl.kernel(
        _body,
        out_type=jax.ShapeDtypeStruct((B, OUT_COLS), jnp.float32),
        mesh=plsc.VectorSubcoreMesh(core_axis_name="c", subcore_axis_name="s"),
        compiler_params=pltpu.CompilerParams(
            use_tc_tiling_on_sc=False, needs_layout_passes=False),
        scratch_types=[
            pltpu.VMEM((CHUNK, NUM_COLS), jnp.float32),
            pltpu.VMEM((IDX_PER_CHUNK,), jnp.int32),
            pltpu.VMEM((IDX_PER_CHUNK, D), jnp.float32),
            pltpu.VMEM((CHUNK, OUT_COLS), jnp.float32),
            pltpu.SemaphoreType.DMA,
        ],
    )
    return run(x, table)


kernel = jax.jit(_run, out_shardings=Format(Layout.AUTO))


# speedup vs baseline: 1.0905x; 1.0905x over previous
import jax
import jax.numpy as jnp
from jax import lax
from jax.experimental import pallas as pl
from jax.experimental.pallas import tpu as pltpu
from jax.experimental.pallas import tpu_sc as plsc

B = 16384
NUM_COLS = 39
N_NUM = 13
N_CAT = 26
D = 32
OUT_COLS = N_NUM + N_CAT * D
CAT_STRIDE = 100000

NC, NS = 2, 16
NW = NC * NS
RW = B // NW
CHUNK = 64
NCHUNK = RW // CHUNK
IDX_PER_CHUNK = CHUNK * N_CAT


def _body(x_ref, table_ref, emb_ref, num_ref, xbuf, idxbuf, gstage, sem):
    wid = lax.axis_index("s") * NC + lax.axis_index("c")
    lanes = lax.iota(jnp.int32, 16)
    zeros = jnp.zeros((16,), jnp.int32)

    def chunk_body(k, carry):
        base = wid * RW + k * CHUNK
        pltpu.sync_copy(x_ref.at[pl.ds(base, CHUNK), :], xbuf)

        r = zeros
        j = lanes
        for s in range(IDX_PER_CHUNK // 16):
            v = plsc.load_gather(xbuf, [r, j + N_NUM])
            idxbuf[pl.ds(s * 16, 16)] = v.astype(jnp.int32) + j * CAT_STRIDE
            t1 = j + 16
            w = t1 >= N_CAT
            r = jnp.where(w, r + 1, r)
            j = jnp.where(w, t1 - N_CAT, t1)

        cp = pltpu.async_copy(table_ref.at[idxbuf.at[:]], gstage.at[:, :], sem)

        pltpu.sync_copy(xbuf.at[:, pl.ds(0, 16)],
                        num_ref.at[pl.ds(base, CHUNK), :])

        cp.wait()
        pltpu.sync_copy(gstage,
                        emb_ref.at[pl.ds(base * N_CAT, IDX_PER_CHUNK), :])
        return carry

    lax.fori_loop(0, NCHUNK, chunk_body, 0)


def _run(x, table):
    run = pl.kernel(
        _body,
        out_type=[
            jax.ShapeDtypeStruct((B * N_CAT, D), jnp.float32),
            jax.ShapeDtypeStruct((B, 16), jnp.float32),
        ],
        mesh=plsc.VectorSubcoreMesh(core_axis_name="c", subcore_axis_name="s"),
        compiler_params=pltpu.CompilerParams(
            use_tc_tiling_on_sc=False, needs_layout_passes=False),
        scratch_types=[
            pltpu.VMEM((CHUNK, NUM_COLS), jnp.float32),
            pltpu.VMEM((IDX_PER_CHUNK,), jnp.int32),
            pltpu.VMEM((IDX_PER_CHUNK, D), jnp.float32),
            pltpu.SemaphoreType.DMA,
        ],
    )
    emb, num = run(x, table)
    return jnp.concatenate([num[:, :N_NUM], emb.reshape(B, N_CAT * D)], axis=1)


kernel = jax.jit(_run)
